# baseline (speedup 1.0000x reference)
import functools

import jax
import jax.numpy as jnp
import numpy as np
from jax import lax
from jax.experimental import pallas as pl
from jax.experimental.pallas import tpu as pltpu
from jax.experimental.pallas import tpu_sc as plsc

_B = 32
_N = 16384
_NP = 2048
_L = 16
_NC = 2
_NS = 16


def _perm_table() -> np.ndarray:
    def build():
        key = jax.random.key(42)
        rows = []
        for i in range(_B):
            k = jax.random.fold_in(key, i)
            rows.append(np.asarray(jax.random.permutation(k, _N))[:_NP])
        return np.stack(rows).astype(np.int32)

    try:
        with jax.default_device(jax.devices("cpu")[0]):
            return build()
    except Exception:
        return build()


_PERMS = _perm_table()


def _sc_body(
    pts_hbm, perm_hbm, out_hbm, x_v, y_v, z_v, perm_v, nz_v, out_v, sem, sem2
):
    b = lax.axis_index("s") * _NC + lax.axis_index("c")
    _H = _N // 2

    cx0 = pltpu.async_copy(pts_hbm.at[0, b, pl.ds(0, _H)], x_v.at[pl.ds(0, _H)], sem)
    cy0 = pltpu.async_copy(pts_hbm.at[1, b, pl.ds(0, _H)], y_v.at[pl.ds(0, _H)], sem)
    cz0 = pltpu.async_copy(pts_hbm.at[2, b, pl.ds(0, _H)], z_v.at[pl.ds(0, _H)], sem)
    cx1 = pltpu.async_copy(pts_hbm.at[0, b, pl.ds(_H, _H)], x_v.at[pl.ds(_H, _H)], sem2)
    cy1 = pltpu.async_copy(pts_hbm.at[1, b, pl.ds(_H, _H)], y_v.at[pl.ds(_H, _H)], sem2)
    cz1 = pltpu.async_copy(pts_hbm.at[2, b, pl.ds(_H, _H)], z_v.at[pl.ds(_H, _H)], sem2)
    cp = pltpu.async_copy(perm_hbm.at[b], perm_v, sem2)
    cx0.wait()
    cy0.wait()
    cz0.wait()

    iota = lax.iota(jnp.int32, _L)

    def mask_step(i, off):
        base = i * _L
        s = x_v[pl.ds(base, _L)] + y_v[pl.ds(base, _L)] + z_v[pl.ds(base, _L)]
        m = s != 0.0
        mi = m.astype(jnp.int32)
        ranks = plsc.cumsum(mi) - mi + off
        plsc.store_scatter(nz_v, [ranks], iota + base, mask=m)
        return off + plsc.all_reduce_population_count(m)

    off_half = plsc.parallel_loop(
        0, _H // _L, carry=jnp.zeros((_L,), jnp.int32), unroll=8
    )(mask_step)

    cx1.wait()
    cy1.wait()
    cz1.wait()
    cp.wait()

    n_pts = plsc.parallel_loop(
        _H // _L, _N // _L, carry=off_half, unroll=8
    )(mask_step)
    nanv = jnp.full((_L,), jnp.nan, jnp.float32)

    @plsc.parallel_loop(0, _NP // _L, unroll=4)
    def sample_loop(t):
        base = t * _L
        pv = perm_v[pl.ds(base, _L)]
        valid = pv < n_pts
        j = plsc.load_gather(nz_v, [pv])
        g = jnp.clip(j, 0, _N - 1)
        x = plsc.load_gather(x_v, [g])
        y = plsc.load_gather(y_v, [g])
        z = plsc.load_gather(z_v, [g])
        out_v[pl.ds(base, _L)] = jnp.where(valid, x, nanv)
        out_v[pl.ds(_NP + base, _L)] = jnp.where(valid, y, nanv)
        out_v[pl.ds(2 * _NP + base, _L)] = jnp.where(valid, z, nanv)

    ox = pltpu.async_copy(out_v.at[pl.ds(0, _NP)], out_hbm.at[0, b], sem)
    oy = pltpu.async_copy(out_v.at[pl.ds(_NP, _NP)], out_hbm.at[1, b], sem)
    oz = pltpu.async_copy(out_v.at[pl.ds(2 * _NP, _NP)], out_hbm.at[2, b], sem)
    ox.wait()
    oy.wait()
    oz.wait()


_sc_sample = functools.partial(
    pl.kernel,
    mesh=plsc.VectorSubcoreMesh(core_axis_name="c", subcore_axis_name="s"),
    compiler_params=pltpu.CompilerParams(
        needs_layout_passes=False, disable_bounds_checks=True
    ),
    out_type=jax.ShapeDtypeStruct((3, _B, _NP), jnp.float32),
    scratch_types=[
        pltpu.VMEM((_N,), jnp.float32),
        pltpu.VMEM((_N,), jnp.float32),
        pltpu.VMEM((_N,), jnp.float32),
        pltpu.VMEM((_NP,), jnp.int32),
        pltpu.VMEM((_N,), jnp.int32),
        pltpu.VMEM((3 * _NP,), jnp.float32),
        pltpu.SemaphoreType.DMA,
        pltpu.SemaphoreType.DMA,
    ],
)(_sc_body)


def kernel(pred_cloud):
    planes = jnp.transpose(pred_cloud, (2, 0, 1))
    out = _sc_sample(planes, jnp.asarray(_PERMS))
    return jnp.transpose(out, (1, 2, 0))


# speedup vs baseline: 1.0170x; 1.0170x over previous
import functools

import jax
import jax.numpy as jnp
import numpy as np
from jax import lax
from jax.experimental import pallas as pl
from jax.experimental.pallas import tpu as pltpu
from jax.experimental.pallas import tpu_sc as plsc

_B = 32
_N = 16384
_NP = 2048
_L = 16
_NC = 2
_NS = 16


def _perm_table() -> np.ndarray:
    def build():
        key = jax.random.key(42)
        rows = []
        for i in range(_B):
            k = jax.random.fold_in(key, i)
            rows.append(np.asarray(jax.random.permutation(k, _N))[:_NP])
        return np.stack(rows).astype(np.int32)

    try:
        with jax.default_device(jax.devices("cpu")[0]):
            return build()
    except Exception:
        return build()


_PERMS = _perm_table()


def _sc_body(
    pts_hbm, perm_hbm, out_hbm, x_v, y_v, z_v, perm_v, nz_v, out_v, sem, psem
):
    b = lax.axis_index("s") * _NC + lax.axis_index("c")

    cx = pltpu.async_copy(pts_hbm.at[0, b], x_v, sem)
    cy = pltpu.async_copy(pts_hbm.at[1, b], y_v, sem)
    cz = pltpu.async_copy(pts_hbm.at[2, b], z_v, sem)
    cp = pltpu.async_copy(perm_hbm.at[b], perm_v, psem)
    cx.wait()
    cy.wait()
    cz.wait()

    iota = lax.iota(jnp.int32, _L)

    @plsc.parallel_loop(0, _N // _L, unroll=8, carry=jnp.zeros((_L,), jnp.int32))
    def mask_loop(i, off):
        base = i * _L
        s = x_v[pl.ds(base, _L)] + y_v[pl.ds(base, _L)] + z_v[pl.ds(base, _L)]
        m = s != 0.0
        mi = m.astype(jnp.int32)
        ranks = plsc.cumsum(mi) - mi + off
        plsc.store_scatter(nz_v, [ranks], iota + base, mask=m)
        return off + plsc.all_reduce_population_count(m)

    n_pts = mask_loop
    cp.wait()
    nanv = jnp.full((_L,), jnp.nan, jnp.float32)

    @plsc.parallel_loop(0, _NP // _L, unroll=4)
    def sample_loop(t):
        base = t * _L
        pv = perm_v[pl.ds(base, _L)]
        valid = pv < n_pts
        j = plsc.load_gather(nz_v, [pv])
        g = jnp.clip(j, 0, _N - 1)
        x = plsc.load_gather(x_v, [g])
        y = plsc.load_gather(y_v, [g])
        z = plsc.load_gather(z_v, [g])
        out_v[pl.ds(base, _L)] = jnp.where(valid, x, nanv)
        out_v[pl.ds(_NP + base, _L)] = jnp.where(valid, y, nanv)
        out_v[pl.ds(2 * _NP + base, _L)] = jnp.where(valid, z, nanv)

    ox = pltpu.async_copy(out_v.at[pl.ds(0, _NP)], out_hbm.at[0, b], sem)
    oy = pltpu.async_copy(out_v.at[pl.ds(_NP, _NP)], out_hbm.at[1, b], sem)
    oz = pltpu.async_copy(out_v.at[pl.ds(2 * _NP, _NP)], out_hbm.at[2, b], sem)
    ox.wait()
    oy.wait()
    oz.wait()


_sc_sample = functools.partial(
    pl.kernel,
    mesh=plsc.VectorSubcoreMesh(core_axis_name="c", subcore_axis_name="s"),
    compiler_params=pltpu.CompilerParams(
        needs_layout_passes=False, disable_bounds_checks=True
    ),
    out_type=jax.ShapeDtypeStruct((3, _B, _NP), jnp.float32),
    scratch_types=[
        pltpu.VMEM((_N,), jnp.float32),
        pltpu.VMEM((_N,), jnp.float32),
        pltpu.VMEM((_N,), jnp.float32),
        pltpu.VMEM((_NP,), jnp.int32),
        pltpu.VMEM((_N,), jnp.int32),
        pltpu.VMEM((3 * _NP,), jnp.float32),
        pltpu.SemaphoreType.DMA,
        pltpu.SemaphoreType.DMA,
    ],
)(_sc_body)


def kernel(pred_cloud):
    planes = jnp.transpose(pred_cloud, (2, 0, 1))
    out = _sc_sample(planes, jnp.asarray(_PERMS))
    return jnp.transpose(out, (1, 2, 0))
